# Initial kernel scaffold; baseline (speedup 1.0000x reference)
import jax
import jax.numpy as jnp
from jax.experimental import pallas as pl


def kernel(idx, W_mean, W_logstd):
    raise NotImplementedError("write your pallas kernel here")


# speedup vs baseline: 1.1469x; 1.1469x over previous
import functools

import jax
import jax.numpy as jnp
from jax import lax
from jax.experimental import pallas as pl
from jax.experimental.pallas import tpu as pltpu
from jax.experimental.pallas import tpu_sc as plsc

D_MODEL = 32
LANES = 16


@functools.partial(jax.jit, static_argnums=())
def _gembed(idx_flat, W_mean, W_logstd):
    B = idx_flat.shape[0]
    info = plsc.get_sparse_core_info()
    NC, NS = info.num_cores, info.num_subcores
    NW = NC * NS
    b_per_w = B // NW
    C = 1024
    n_chunks = b_per_w // C
    assert b_per_w % C == 0

    mesh = plsc.VectorSubcoreMesh(core_axis_name="c", subcore_axis_name="s")

    @functools.partial(
        pl.kernel,
        mesh=mesh,
        compiler_params=pltpu.CompilerParams(use_tc_tiling_on_sc=False),
        out_type=[
            jax.ShapeDtypeStruct((B, D_MODEL), jnp.float32),
            jax.ShapeDtypeStruct((B, D_MODEL), jnp.float32),
        ],
        scratch_types=[
            pltpu.VMEM((b_per_w,), jnp.int32),
            pltpu.VMEM((C, D_MODEL), jnp.float32),
            pltpu.VMEM((C, D_MODEL), jnp.float32),
            pltpu.SemaphoreType.DMA,
            pltpu.SemaphoreType.DMA,
        ],
    )
    def k(idx_hbm, wm_hbm, ws_hbm, mean_hbm, std_hbm,
          idx_v, rows_m, rows_s, sem_m, sem_s):
        wid = lax.axis_index("s") * NC + lax.axis_index("c")
        base = pl.multiple_of(wid * b_per_w, 8)
        pltpu.sync_copy(idx_hbm.at[pl.ds(base, b_per_w)], idx_v)

        def chunk(ci, carry):
            off = pl.multiple_of(ci * C, 8)
            cm = pltpu.async_copy(wm_hbm.at[idx_v.at[pl.ds(off, C)]],
                                  rows_m, sem_m)
            cs = pltpu.async_copy(ws_hbm.at[idx_v.at[pl.ds(off, C)]],
                                  rows_s, sem_s)
            cm.wait()
            pltpu.sync_copy(rows_m, mean_hbm.at[pl.ds(base + off, C)])
            cs.wait()

            def expbody(i, c2):
                for r in range(4):
                    row = i * 4 + r
                    for h in range(D_MODEL // LANES):
                        sl = (row, pl.ds(h * LANES, LANES))
                        rows_s[sl] = jnp.exp(rows_s[sl])
                return c2

            lax.fori_loop(0, C // 4, expbody, 0)
            pltpu.sync_copy(rows_s, std_hbm.at[pl.ds(base + off, C)])
            return carry

        lax.fori_loop(0, n_chunks, chunk, 0)

    return k(idx_flat, W_mean, W_logstd)


def kernel(idx, W_mean, W_logstd):
    B0, H = idx.shape
    idx_flat = idx.reshape(B0 * H).astype(jnp.int32)
    mean_flat, std_flat = _gembed(idx_flat, W_mean, W_logstd)
    return (mean_flat.reshape(B0, H, D_MODEL), std_flat.reshape(B0, H, D_MODEL))


# speedup vs baseline: 1.1738x; 1.0235x over previous
import functools

import jax
import jax.numpy as jnp
from jax import lax
from jax.experimental import pallas as pl
from jax.experimental.pallas import tpu as pltpu
from jax.experimental.pallas import tpu_sc as plsc

D_MODEL = 32
LANES = 16
NBUF = 4
C = 400


@jax.jit
def _gembed(idx_flat, W_mean, W_logstd):
    B = idx_flat.shape[0]
    info = plsc.get_sparse_core_info()
    NC, NS = info.num_cores, info.num_subcores
    NW = NC * NS
    b_per_w = B // NW
    n_chunks = b_per_w // C
    n_grp = n_chunks // NBUF
    assert b_per_w % (C * NBUF) == 0 and C % 8 == 0

    mesh = plsc.VectorSubcoreMesh(core_axis_name="c", subcore_axis_name="s")

    scratch = (
        [pltpu.VMEM((b_per_w,), jnp.int32)]
        + [pltpu.VMEM((C, D_MODEL), jnp.float32) for _ in range(2 * NBUF)]
        + [pltpu.SemaphoreType.DMA for _ in range(4 * NBUF)]
    )

    @functools.partial(
        pl.kernel,
        mesh=mesh,
        compiler_params=pltpu.CompilerParams(use_tc_tiling_on_sc=False),
        out_type=[
            jax.ShapeDtypeStruct((B, D_MODEL), jnp.float32),
            jax.ShapeDtypeStruct((B, D_MODEL), jnp.float32),
        ],
        scratch_types=scratch,
    )
    def k(idx_hbm, wm_hbm, ws_hbm, mean_hbm, std_hbm, idx_v, *rest):
        rows_m = rest[0:NBUF]
        rows_s = rest[NBUF:2 * NBUF]
        sem_gm = rest[2 * NBUF:3 * NBUF]
        sem_gs = rest[3 * NBUF:4 * NBUF]
        sem_wm = rest[4 * NBUF:5 * NBUF]
        sem_ws = rest[5 * NBUF:6 * NBUF]

        wid = lax.axis_index("s") * NC + lax.axis_index("c")
        base = pl.multiple_of(wid * b_per_w, 8)
        pltpu.sync_copy(idx_hbm.at[pl.ds(base, b_per_w)], idx_v)

        def start_gathers(kc, b):
            off = pl.multiple_of(kc * C, 8)
            pltpu.async_copy(wm_hbm.at[idx_v.at[pl.ds(off, C)]], rows_m[b],
                             sem_gm[b])
            pltpu.async_copy(ws_hbm.at[idx_v.at[pl.ds(off, C)]], rows_s[b],
                             sem_gs[b])

        def wait_writes(b):
            pltpu.make_async_copy(rows_m[b], mean_hbm.at[pl.ds(base, C)],
                                  sem_wm[b]).wait()
            pltpu.make_async_copy(rows_s[b], std_hbm.at[pl.ds(base, C)],
                                  sem_ws[b]).wait()

        for b in range(NBUF):
            start_gathers(b, b)

        @pl.loop(0, n_grp)
        def grp_loop(grp):
            for b in range(NBUF):
                kc = grp * NBUF + b
                off = pl.multiple_of(kc * C, 8)
                pltpu.make_async_copy(
                    wm_hbm.at[idx_v.at[pl.ds(off, C)]], rows_m[b],
                    sem_gm[b]).wait()
                pltpu.async_copy(rows_m[b], mean_hbm.at[pl.ds(base + off, C)],
                                 sem_wm[b])
                pltpu.make_async_copy(
                    ws_hbm.at[idx_v.at[pl.ds(off, C)]], rows_s[b],
                    sem_gs[b]).wait()

                rs = rows_s[b]

                @plsc.parallel_loop(0, C, step=8)
                def expbody(i):
                    for r in range(8):
                        for h in range(D_MODEL // LANES):
                            sl = (i + r, pl.ds(h * LANES, LANES))
                            rs[sl] = jnp.exp(rs[sl])

                pltpu.async_copy(rows_s[b], std_hbm.at[pl.ds(base + off, C)],
                                 sem_ws[b])

            @pl.when(grp < n_grp - 1)
            def _():
                for b in range(NBUF):
                    wait_writes(b)
                    start_gathers((grp + 1) * NBUF + b, b)

        for b in range(NBUF):
            wait_writes(b)

    return k(idx_flat, W_mean, W_logstd)


def kernel(idx, W_mean, W_logstd):
    B0, H = idx.shape
    idx_flat = idx.reshape(B0 * H).astype(jnp.int32)
    mean_flat, std_flat = _gembed(idx_flat, W_mean, W_logstd)
    return (mean_flat.reshape(B0, H, D_MODEL), std_flat.reshape(B0, H, D_MODEL))


# speedup vs baseline: 1.8967x; 1.6158x over previous
import functools

import jax
import jax.numpy as jnp
from jax import lax
from jax.experimental import pallas as pl
from jax.experimental.pallas import tpu as pltpu
from jax.experimental.pallas import tpu_sc as plsc

D_MODEL = 32
LANES = 16
NBUF = 4
C = 160


@jax.jit
def _gembed(idx_flat, W_mean, W_logstd):
    B = idx_flat.shape[0]
    info = plsc.get_sparse_core_info()
    NC, NS = info.num_cores, info.num_subcores
    NW = NC * NS
    b_per_w = B // NW
    n_chunks = b_per_w // C
    n_grp = n_chunks // NBUF
    CE = C * D_MODEL
    assert b_per_w % (C * NBUF) == 0 and C % 8 == 0

    mesh = plsc.VectorSubcoreMesh(core_axis_name="c", subcore_axis_name="s")

    scratch = (
        [pltpu.VMEM((b_per_w,), jnp.int32)]
        + [pltpu.VMEM((C, D_MODEL), jnp.float32) for _ in range(2 * NBUF)]
        + [pltpu.VMEM((CE,), jnp.float32) for _ in range(2 * NBUF)]
        + [pltpu.SemaphoreType.DMA for _ in range(4 * NBUF)]
    )

    @functools.partial(
        pl.kernel,
        mesh=mesh,
        compiler_params=pltpu.CompilerParams(use_tc_tiling_on_sc=False),
        out_type=[
            jax.ShapeDtypeStruct((B * D_MODEL,), jnp.float32),
            jax.ShapeDtypeStruct((B * D_MODEL,), jnp.float32),
        ],
        scratch_types=scratch,
    )
    def k(idx_hbm, wm_hbm, ws_hbm, mean_hbm, std_hbm, idx_v, *rest):
        gbuf_m = rest[0:NBUF]
        gbuf_s = rest[NBUF:2 * NBUF]
        st_m = rest[2 * NBUF:3 * NBUF]
        st_s = rest[3 * NBUF:4 * NBUF]
        sem_gm = rest[4 * NBUF:5 * NBUF]
        sem_gs = rest[5 * NBUF:6 * NBUF]
        sem_wm = rest[6 * NBUF:7 * NBUF]
        sem_ws = rest[7 * NBUF:8 * NBUF]

        wid = lax.axis_index("s") * NC + lax.axis_index("c")
        base = pl.multiple_of(wid * b_per_w, 8)
        ebase = pl.multiple_of(wid * b_per_w * D_MODEL, 8)
        pltpu.sync_copy(idx_hbm.at[pl.ds(base, b_per_w)], idx_v)

        def start_gathers(kc, b):
            off = pl.multiple_of(kc * C, 8)
            pltpu.async_copy(wm_hbm.at[idx_v.at[pl.ds(off, C)]], gbuf_m[b],
                             sem_gm[b])
            pltpu.async_copy(ws_hbm.at[idx_v.at[pl.ds(off, C)]], gbuf_s[b],
                             sem_gs[b])

        for b in range(NBUF):
            start_gathers(b, b)

        @pl.loop(0, n_grp)
        def grp_loop(grp):
            for b in range(NBUF):
                kc = grp * NBUF + b
                eoff = pl.multiple_of(kc * CE, 8)

                pltpu.make_async_copy(wm_hbm.at[idx_v.at[pl.ds(0, C)]],
                                      gbuf_m[b], sem_gm[b]).wait()

                @pl.when(grp > 0)
                def _():
                    pltpu.make_async_copy(st_m[b],
                                          mean_hbm.at[pl.ds(ebase, CE)],
                                          sem_wm[b]).wait()

                gm, sm = gbuf_m[b], st_m[b]

                @plsc.parallel_loop(0, C, step=4)
                def repack_m(i):
                    for r in range(4):
                        for h in range(D_MODEL // LANES):
                            sm[pl.ds((i + r) * D_MODEL + h * LANES, LANES)] = \
                                gm[i + r, pl.ds(h * LANES, LANES)]

                pltpu.async_copy(st_m[b], mean_hbm.at[pl.ds(ebase + eoff, CE)],
                                 sem_wm[b])

                pltpu.make_async_copy(ws_hbm.at[idx_v.at[pl.ds(0, C)]],
                                      gbuf_s[b], sem_gs[b]).wait()

                @pl.when(grp > 0)
                def _():
                    pltpu.make_async_copy(st_s[b],
                                          std_hbm.at[pl.ds(ebase, CE)],
                                          sem_ws[b]).wait()

                gs, ss = gbuf_s[b], st_s[b]

                @plsc.parallel_loop(0, C, step=4)
                def repack_s(i):
                    for r in range(4):
                        for h in range(D_MODEL // LANES):
                            ss[pl.ds((i + r) * D_MODEL + h * LANES, LANES)] = \
                                jnp.exp(gs[i + r, pl.ds(h * LANES, LANES)])

                pltpu.async_copy(st_s[b], std_hbm.at[pl.ds(ebase + eoff, CE)],
                                 sem_ws[b])

                @pl.when(grp < n_grp - 1)
                def _():
                    start_gathers((grp + 1) * NBUF + b, b)

        for b in range(NBUF):
            pltpu.make_async_copy(st_m[b], mean_hbm.at[pl.ds(ebase, CE)],
                                  sem_wm[b]).wait()
            pltpu.make_async_copy(st_s[b], std_hbm.at[pl.ds(ebase, CE)],
                                  sem_ws[b]).wait()

    return k(idx_flat, W_mean, W_logstd)


def kernel(idx, W_mean, W_logstd):
    B0, H = idx.shape
    idx_flat = idx.reshape(B0 * H).astype(jnp.int32)
    mean_flat, std_flat = _gembed(idx_flat, W_mean, W_logstd)
    return (mean_flat.reshape(B0, H, D_MODEL), std_flat.reshape(B0, H, D_MODEL))
